# baseline (speedup 1.0000x reference)
import functools

import jax
import jax.numpy as jnp
from jax import lax
from jax.experimental import pallas as pl
from jax.experimental.pallas import tpu as pltpu
from jax.experimental.pallas import tpu_sc as plsc

VOCAB = 100
D = 128
B = 16384
L = 16
NC = 2
NS = 16
NW = NC * NS
BPW = B // NW
C = 64
NCH = BPW // C
TROWS = 4 * VOCAB


def _body(yt_hbm, wcat_hbm, out_hbm,
          idx_v, tab_v, acc0, acc1, so0, so1):
    wid = lax.axis_index("s") * NC + lax.axis_index("c")
    base = wid * BPW

    pltpu.sync_copy(yt_hbm.at[pl.ds(wid * 4 * BPW, 4 * BPW)], idx_v)
    pltpu.sync_copy(wcat_hbm, tab_v)

    accs = (acc0, acc1)
    osems = (so0, so1)

    @pl.loop(0, NCH, step=2)
    def chunk_pair(ci0):
        for b in range(2):
            ci = ci0 + b
            acc = accs[b]

            @pl.when(ci0 > 0)
            def _():
                pltpu.make_async_copy(
                    acc, out_hbm.at[pl.ds(0, C)], osems[b]).wait()

            with jax.named_scope("rowsum"):
                @plsc.parallel_loop(0, C // 4, unroll=2)
                def row_sum(r4):
                    r = r4 * 4
                    iv = idx_v[pl.ds((ci * C + r) * 4, L)]
                    for q in range(4):
                        for k in range(D // L):
                            off = k * L
                            a = tab_v[iv[4 * q], pl.ds(off, L)]
                            a = a + tab_v[iv[4 * q + 1], pl.ds(off, L)]
                            a = a + tab_v[iv[4 * q + 2], pl.ds(off, L)]
                            a = a + tab_v[iv[4 * q + 3], pl.ds(off, L)]
                            acc[r + q, pl.ds(off, L)] = a

            pltpu.async_copy(
                acc, out_hbm.at[pl.ds(base + ci * C, C)], osems[b])

    for b in range(2):
        pltpu.make_async_copy(
            accs[b], out_hbm.at[pl.ds(0, C)], osems[b]).wait()


_emb = functools.partial(
    pl.kernel,
    mesh=plsc.VectorSubcoreMesh(core_axis_name="c", subcore_axis_name="s"),
    compiler_params=pltpu.CompilerParams(needs_layout_passes=False),
    out_type=jax.ShapeDtypeStruct((B, D), jnp.float32),
    scratch_types=[
        pltpu.VMEM((4 * BPW,), jnp.int32),
        pltpu.VMEM((TROWS, D), jnp.float32),
        pltpu.VMEM((C, D), jnp.float32),
        pltpu.VMEM((C, D), jnp.float32),
        pltpu.SemaphoreType.DMA,
        pltpu.SemaphoreType.DMA,
    ],
)(_body)


@jax.jit
def kernel(y, W0, W1, W2, W3):
    wcat = jnp.concatenate([W0, W1, W2, W3], axis=0)
    yb = (y + jnp.arange(4, dtype=y.dtype) * VOCAB).reshape(-1)
    return _emb(yb, wcat)


# speedup vs baseline: 1.2254x; 1.2254x over previous
import functools

import jax
import jax.numpy as jnp
from jax import lax
from jax.experimental import pallas as pl
from jax.experimental.pallas import tpu as pltpu
from jax.experimental.pallas import tpu_sc as plsc

VOCAB = 100
D = 128
B = 16384
L = 16
NC = 2
NS = 16
NW = NC * NS
BPW = B // NW
C = 64
NCH = BPW // C
TROWS = 512


def _body(yt_hbm, wcat_hbm, out_hbm,
          idx_v, gbuf, acc0, acc1, tab_sh, sg0, sg1, so0, so1):
    sid = lax.axis_index("s")
    wid = sid * NC + lax.axis_index("c")
    base = wid * BPW

    pltpu.sync_copy(yt_hbm.at[pl.ds(wid * 4 * BPW, 4 * BPW)], idx_v)

    rows_per_tile = TROWS // NS
    pltpu.sync_copy(wcat_hbm.at[pl.ds(sid * rows_per_tile, rows_per_tile)],
                    tab_sh.at[pl.ds(sid * rows_per_tile, rows_per_tile)])

    plsc.subcore_barrier()

    accs = (acc0, acc1)
    gsems = (sg0, sg1)
    osems = (so0, so1)

    def fire(ci, slot):
        return [
            pltpu.async_copy(
                tab_sh.at[idx_v.at[pl.ds(ci * 4 * C + h * 2 * C, 2 * C)]],
                gbuf.at[slot, h], gsems[slot])
            for h in range(2)
        ]

    pend_g = fire(0, 0)
    pend_o = [None, None]
    for ci in range(NCH):
        slot = ci % 2
        nxt = fire(ci + 1, 1 - slot) if ci + 1 < NCH else []
        with jax.named_scope("gwait"):
            for cp in pend_g:
                cp.wait()
            pend_g = nxt
            if pend_o[slot] is not None:
                pend_o[slot].wait()

        acc = accs[slot]

        with jax.named_scope("adds"):
            @plsc.parallel_loop(0, C, unroll=8)
            def batch_el(b):
                for k in range(D // L):
                    off = k * L
                    a = gbuf[slot, 0, b, pl.ds(off, L)]
                    a = a + gbuf[slot, 0, C + b, pl.ds(off, L)]
                    a = a + gbuf[slot, 1, b, pl.ds(off, L)]
                    a = a + gbuf[slot, 1, C + b, pl.ds(off, L)]
                    acc[b, pl.ds(off, L)] = a

        pend_o[slot] = pltpu.async_copy(
            acc, out_hbm.at[pl.ds(base + ci * C, C)], osems[slot])

    for po in pend_o:
        if po is not None:
            po.wait()


_emb = functools.partial(
    pl.kernel,
    mesh=plsc.VectorSubcoreMesh(core_axis_name="c", subcore_axis_name="s"),
    compiler_params=pltpu.CompilerParams(needs_layout_passes=False),
    out_type=jax.ShapeDtypeStruct((B, D), jnp.float32),
    scratch_types=[
        pltpu.VMEM((4 * BPW,), jnp.int32),
        pltpu.VMEM((2, 2, 2 * C, D), jnp.float32),
        pltpu.VMEM((C, D), jnp.float32),
        pltpu.VMEM((C, D), jnp.float32),
        pltpu.VMEM_SHARED((TROWS, D), jnp.float32),
        pltpu.SemaphoreType.DMA,
        pltpu.SemaphoreType.DMA,
        pltpu.SemaphoreType.DMA,
        pltpu.SemaphoreType.DMA,
    ],
)(_body)


@jax.jit
def kernel(y, W0, W1, W2, W3):
    wcat = jnp.concatenate([W0, W1, W2, W3], axis=0)
    wcat = jnp.pad(wcat, ((0, TROWS - 4 * VOCAB), (0, 0)))
    yb = (y + jnp.arange(4, dtype=y.dtype) * VOCAB)
    yb = yb.reshape(NW, NCH, C, 4).transpose(0, 1, 3, 2).reshape(-1)
    return _emb(yb, wcat)


# speedup vs baseline: 1.7380x; 1.4182x over previous
import functools

import jax
import jax.numpy as jnp
from jax import lax
from jax.experimental import pallas as pl
from jax.experimental.pallas import tpu as pltpu
from jax.experimental.pallas import tpu_sc as plsc

VOCAB = 100
D = 128
B = 16384
L = 16
NC = 2
NS = 16
NW = NC * NS
BPW = B // NW
C = 64
NCH = BPW // C
TROWS = 512


def _body(yt_hbm, wcat_hbm, out_hbm,
          idx_v, gbuf, acc0, acc1, acc2, tab_sh,
          sg0, sg1, sg2, so0, so1, so2):
    sid = lax.axis_index("s")
    wid = sid * NC + lax.axis_index("c")
    base = wid * BPW

    pltpu.sync_copy(yt_hbm.at[pl.ds(wid * 4 * BPW, 4 * BPW)], idx_v)

    rows_per_tile = TROWS // NS
    pltpu.sync_copy(wcat_hbm.at[pl.ds(sid * rows_per_tile, rows_per_tile)],
                    tab_sh.at[pl.ds(sid * rows_per_tile, rows_per_tile)])

    plsc.subcore_barrier()

    accs = (acc0, acc1, acc2)
    gsems = (sg0, sg1, sg2)
    osems = (so0, so1, so2)
    NSL = 3

    def fire(ci):
        slot = ci % NSL
        return [
            pltpu.async_copy(
                tab_sh.at[idx_v.at[pl.ds(ci * 4 * C + h * 2 * C, 2 * C)]],
                gbuf.at[slot, h], gsems[slot])
            for h in range(2)
        ]

    pend_g = [fire(0), fire(1), None]
    pend_o = [None, None, None]
    for ci in range(NCH):
        slot = ci % NSL
        if ci + 2 < NCH:
            pend_g[(ci + 2) % NSL] = fire(ci + 2)
        with jax.named_scope("gwait"):
            for cp in pend_g[slot]:
                cp.wait()
            if pend_o[slot] is not None:
                pend_o[slot].wait()

        acc = accs[slot]

        with jax.named_scope("adds"):
            @plsc.parallel_loop(0, C, unroll=2)
            def batch_el(b):
                for k in range(D // L):
                    off = k * L
                    a = gbuf[slot, 0, b, pl.ds(off, L)]
                    a = a + gbuf[slot, 0, C + b, pl.ds(off, L)]
                    a = a + gbuf[slot, 1, b, pl.ds(off, L)]
                    a = a + gbuf[slot, 1, C + b, pl.ds(off, L)]
                    acc[b, pl.ds(off, L)] = a

        pend_o[slot] = pltpu.async_copy(
            acc, out_hbm.at[pl.ds(base + ci * C, C)], osems[slot])

    for po in pend_o:
        if po is not None:
            po.wait()


_emb = functools.partial(
    pl.kernel,
    mesh=plsc.VectorSubcoreMesh(core_axis_name="c", subcore_axis_name="s"),
    compiler_params=pltpu.CompilerParams(needs_layout_passes=False),
    out_type=jax.ShapeDtypeStruct((B, D), jnp.float32),
    scratch_types=[
        pltpu.VMEM((4 * BPW,), jnp.int32),
        pltpu.VMEM((3, 2, 2 * C, D), jnp.float32),
        pltpu.VMEM((C, D), jnp.float32),
        pltpu.VMEM((C, D), jnp.float32),
        pltpu.VMEM((C, D), jnp.float32),
        pltpu.VMEM_SHARED((TROWS, D), jnp.float32),
        pltpu.SemaphoreType.DMA,
        pltpu.SemaphoreType.DMA,
        pltpu.SemaphoreType.DMA,
        pltpu.SemaphoreType.DMA,
        pltpu.SemaphoreType.DMA,
        pltpu.SemaphoreType.DMA,
    ],
)(_body)


@jax.jit
def kernel(y, W0, W1, W2, W3):
    wcat = jnp.concatenate([W0, W1, W2, W3], axis=0)
    wcat = jnp.pad(wcat, ((0, TROWS - 4 * VOCAB), (0, 0)))
    yb = (y + jnp.arange(4, dtype=y.dtype) * VOCAB)
    yb = yb.reshape(NW, NCH, C, 4).transpose(0, 1, 3, 2).reshape(-1)
    return _emb(yb, wcat)
